# baseline (speedup 1.0000x reference)
import jax
import jax.numpy as jnp
from jax import lax
from jax.experimental import pallas as pl
from jax.experimental.pallas import tpu as pltpu
from jax.experimental.pallas import tpu_sc as plsc

_B = 16384
_D = 128
_NREL = 1000
_L = 16
_NW = 32

_BSC = 12288
_BTC = _B - _BSC
_BPW = _BSC // _NW
_C = 128
_NCHUNK = _BPW // _C
_TBLK = 512


def _sc_body(h_hbm, t_hbm, r_hbm, tab_hbm, out_hbm,
             idx_v, h_v, t_v, rel_v, out_v,
             isem, sem0, sem1, osem):
    cid = lax.axis_index("c")
    sid = lax.axis_index("s")
    wid = sid * 2 + cid
    base = wid * _BPW
    lane = lax.iota(jnp.int32, _L)
    last = lane == (_L - 1)
    sems = (sem0, sem1)

    idx_copies = [
        pltpu.async_copy(r_hbm.at[pl.ds(base + c * _C, _C)], idx_v.at[c], isem)
        for c in range(_NCHUNK)
    ]
    for cp in idx_copies:
        cp.wait()

    def fetch(c, b):
        off = base + c * _C
        return [
            pltpu.async_copy(tab_hbm.at[idx_v.at[c]], rel_v.at[b], sems[b]),
            pltpu.async_copy(h_hbm.at[pl.ds(off, _C), :], h_v.at[b], sems[b]),
            pltpu.async_copy(t_hbm.at[pl.ds(off, _C), :], t_v.at[b], sems[b]),
        ]

    pending = fetch(0, 0)
    out_copies = [None, None]
    for c in range(_NCHUNK):
        b = c % 2
        nxt = fetch(c + 1, 1 - b) if c + 1 < _NCHUNK else None
        for cp in pending:
            cp.wait()
        pending = nxt
        if out_copies[b] is not None:
            out_copies[b].wait()

        @plsc.parallel_loop(0, _C, unroll=4)
        def row_body(i, _b=b):
            accs = [jnp.zeros((_L,), jnp.float32) for _ in range(4)]
            for j in range(_D // _L):
                sl = pl.ds(j * _L, _L)
                accs[j % 4] = accs[j % 4] + (
                    h_v[_b, i, sl] * rel_v[_b, i, sl]) * t_v[_b, i, sl]
            acc = (accs[0] + accs[1]) + (accs[2] + accs[3])
            cum = plsc.cumsum(acc)
            plsc.store_compressed(out_v.at[_b, pl.ds(i, _L)], cum, mask=last)

        out_copies[b] = pltpu.async_copy(
            out_v.at[b, pl.ds(0, _C)], out_hbm.at[pl.ds(base + c * _C, _C)],
            osem)
    for cp in out_copies:
        if cp is not None:
            cp.wait()


def _make_sc():
    mesh = plsc.VectorSubcoreMesh(core_axis_name="c", subcore_axis_name="s")
    return pl.kernel(
        _sc_body,
        out_type=jax.ShapeDtypeStruct((_BSC,), jnp.float32),
        mesh=mesh,
        compiler_params=pltpu.CompilerParams(needs_layout_passes=False),
        scratch_types=[
            pltpu.VMEM((_NCHUNK, _C), jnp.int32),
            pltpu.VMEM((2, _C, _D), jnp.float32),
            pltpu.VMEM((2, _C, _D), jnp.float32),
            pltpu.VMEM((2, _C, _D), jnp.float32),
            pltpu.VMEM((2, _C + _L), jnp.float32),
            pltpu.SemaphoreType.DMA,
            pltpu.SemaphoreType.DMA,
            pltpu.SemaphoreType.DMA,
            pltpu.SemaphoreType.DMA,
        ],
    )


def _tc_body(h_ref, t_ref, r_ref, rel_ref, out_ref):
    p = h_ref[...] * t_ref[...]
    s = lax.dot_general(p, rel_ref[...], (((1,), (1,)), ((), ())),
                        precision=lax.Precision.HIGHEST,
                        preferred_element_type=jnp.float32)
    cols = lax.broadcasted_iota(jnp.int32, (_TBLK, _NREL), 1)
    sel = jnp.where(cols == r_ref[...][:, None], s, 0.0)
    out_ref[...] = jnp.sum(sel, axis=1)


def _make_tc():
    nblk = _BTC // _TBLK
    off = _BSC // _TBLK
    return pl.pallas_call(
        _tc_body,
        grid=(nblk,),
        in_specs=[
            pl.BlockSpec((_TBLK, _D), lambda i: (i + off, 0)),
            pl.BlockSpec((_TBLK, _D), lambda i: (i + off, 0)),
            pl.BlockSpec((_TBLK,), lambda i: (i + off,)),
            pl.BlockSpec((_NREL, _D), lambda i: (0, 0)),
        ],
        out_specs=pl.BlockSpec((_TBLK,), lambda i: (i,)),
        out_shape=jax.ShapeDtypeStruct((_BTC,), jnp.float32),
    )


@jax.jit
def _dist_mult(h, t, r, rel_emb):
    sc_out = _make_sc()(h, t, r, rel_emb)
    tc_out = _make_tc()(h, t, r, rel_emb)
    return jnp.concatenate([sc_out, tc_out])


def kernel(h, t, r, rel_emb):
    return _dist_mult(h, t, r.astype(jnp.int32), rel_emb)


# speedup vs baseline: 1.0877x; 1.0877x over previous
import jax
import jax.numpy as jnp
from jax import lax
from jax.experimental import pallas as pl
from jax.experimental.pallas import tpu as pltpu
from jax.experimental.pallas import tpu_sc as plsc

_B = 16384
_D = 128
_NREL = 1000
_L = 16
_NW = 32

_BSC = 12288
_BTC = _B - _BSC
_BPW = _BSC // _NW
_C = 128
_NCHUNK = _BPW // _C
_TBLK = 512


def _sc_body(h_hbm, t_hbm, r_hbm, tab_hbm, out_hbm,
             idx_v, h_v, t_v, rel_v, out_v,
             isem, sem0, sem1, osem):
    cid = lax.axis_index("c")
    sid = lax.axis_index("s")
    wid = sid * 2 + cid
    base = wid * _BPW
    lane = lax.iota(jnp.int32, _L)
    last = lane == (_L - 1)
    sems = (sem0, sem1)

    idx_copies = [
        pltpu.async_copy(r_hbm.at[pl.ds(base + c * _C, _C)], idx_v.at[c], isem)
        for c in range(_NCHUNK)
    ]
    for cp in idx_copies:
        cp.wait()

    def fetch(c, b):
        off = base + c * _C
        return [
            pltpu.async_copy(tab_hbm.at[idx_v.at[c]], rel_v.at[b], sems[b]),
            pltpu.async_copy(h_hbm.at[pl.ds(off, _C), :], h_v.at[b], sems[b]),
            pltpu.async_copy(t_hbm.at[pl.ds(off, _C), :], t_v.at[b], sems[b]),
        ]

    pending = fetch(0, 0)
    out_copies = [None, None]
    for c in range(_NCHUNK):
        b = c % 2
        nxt = fetch(c + 1, 1 - b) if c + 1 < _NCHUNK else None
        for cp in pending:
            cp.wait()
        pending = nxt
        if out_copies[b] is not None:
            out_copies[b].wait()

        @plsc.parallel_loop(0, _C, unroll=4)
        def row_body(i, _b=b):
            accs = [jnp.zeros((_L,), jnp.float32) for _ in range(4)]
            for j in range(_D // _L):
                sl = pl.ds(j * _L, _L)
                accs[j % 4] = accs[j % 4] + (
                    h_v[_b, i, sl] * rel_v[_b, i, sl]) * t_v[_b, i, sl]
            acc = (accs[0] + accs[1]) + (accs[2] + accs[3])
            cum = plsc.cumsum(acc)
            plsc.store_compressed(out_v.at[_b, pl.ds(i, _L)], cum, mask=last)

        out_copies[b] = pltpu.async_copy(
            out_v.at[b, pl.ds(0, _C)], out_hbm.at[pl.ds(base + c * _C, _C)],
            osem)
    for cp in out_copies:
        if cp is not None:
            cp.wait()


def _make_sc():
    mesh = plsc.VectorSubcoreMesh(core_axis_name="c", subcore_axis_name="s")
    return pl.kernel(
        _sc_body,
        out_type=jax.ShapeDtypeStruct((_BSC,), jnp.float32),
        mesh=mesh,
        compiler_params=pltpu.CompilerParams(needs_layout_passes=False),
        scratch_types=[
            pltpu.VMEM((_NCHUNK, _C), jnp.int32),
            pltpu.VMEM((2, _C, _D), jnp.float32),
            pltpu.VMEM((2, _C, _D), jnp.float32),
            pltpu.VMEM((2, _C, _D), jnp.float32),
            pltpu.VMEM((2, _C + _L), jnp.float32),
            pltpu.SemaphoreType.DMA,
            pltpu.SemaphoreType.DMA,
            pltpu.SemaphoreType.DMA,
            pltpu.SemaphoreType.DMA,
        ],
    )


def _dot_t(a, b):
    return lax.dot_general(a, b, (((1,), (1,)), ((), ())),
                           preferred_element_type=jnp.float32)


def _tc_body(h_ref, t_ref, r_ref, rel_ref, out_ref):
    p = h_ref[...] * t_ref[...]
    rel = rel_ref[...]
    p_hi = p.astype(jnp.bfloat16)
    p_lo = (p - p_hi.astype(jnp.float32)).astype(jnp.bfloat16)
    rel_hi = rel.astype(jnp.bfloat16)
    rel_lo = (rel - rel_hi.astype(jnp.float32)).astype(jnp.bfloat16)
    s = _dot_t(p_hi, rel_hi) + (_dot_t(p_hi, rel_lo) + _dot_t(p_lo, rel_hi))
    cols = lax.broadcasted_iota(jnp.int32, (_TBLK, _NREL), 1)
    sel = jnp.where(cols == r_ref[...][:, None], s, 0.0)
    out_ref[...] = jnp.sum(sel, axis=1)


def _make_tc():
    nblk = _BTC // _TBLK
    off = _BSC // _TBLK
    return pl.pallas_call(
        _tc_body,
        grid=(nblk,),
        in_specs=[
            pl.BlockSpec((_TBLK, _D), lambda i: (i + off, 0)),
            pl.BlockSpec((_TBLK, _D), lambda i: (i + off, 0)),
            pl.BlockSpec((_TBLK,), lambda i: (i + off,)),
            pl.BlockSpec((_NREL, _D), lambda i: (0, 0)),
        ],
        out_specs=pl.BlockSpec((_TBLK,), lambda i: (i,)),
        out_shape=jax.ShapeDtypeStruct((_BTC,), jnp.float32),
    )


@jax.jit
def _dist_mult(h, t, r, rel_emb):
    sc_out = _make_sc()(h, t, r, rel_emb)
    tc_out = _make_tc()(h, t, r, rel_emb)
    return jnp.concatenate([sc_out, tc_out])


def kernel(h, t, r, rel_emb):
    return _dist_mult(h, t, r.astype(jnp.int32), rel_emb)
